# baseline (speedup 1.0000x reference)
import functools

import jax
import jax.numpy as jnp
from jax import lax
from jax.experimental import pallas as pl
from jax.experimental.pallas import tpu as pltpu
from jax.experimental.pallas import tpu_sc as plsc

EMBED_DIM = 4096
N_EXPERTS = 64
TOP_K = 8
BT = 1024

_INFO = plsc.get_sparse_core_info()
NC = _INFO.num_cores
NS = _INFO.num_subcores
L = _INFO.num_lanes
NW = NC * NS


def _matmul_body(x_ref, w_ref, logits_ref):
    logits_ref[...] = jax.lax.dot_general(
        w_ref[...], x_ref[...], (((1,), (1,)), ((), ())),
        preferred_element_type=jnp.float32)


def _route_body(tpw, ng, logits_hbm, probs_hbm, idx_hbm, stats_hbm,
                ltile, etile, ptile, itile, accp, accm):
    wid = lax.axis_index("s") * NC + lax.axis_index("c")
    base = wid * tpw
    pltpu.sync_copy(logits_hbm.at[:, pl.ds(base, tpw)], ltile)

    zero = jnp.zeros((L,), jnp.float32)
    neginf = jnp.full((L,), -jnp.inf, jnp.float32)

    def _zero_acc(e, c):
        accp[e, :] = zero
        accm[e, :] = zero
        return c
    lax.fori_loop(0, N_EXPERTS, _zero_acc, 0, unroll=8)

    NCH = 4
    CH = N_EXPERTS // NCH

    def _group(g, c):
        t0 = g * L
        ts = pl.ds(t0, L)

        def _mx(i, ms):
            return tuple(
                jnp.maximum(ms[j], ltile[j * CH + i, ts]) for j in range(NCH))
        ms = lax.fori_loop(0, CH, _mx, (neginf,) * NCH, unroll=4)
        m1 = jnp.maximum(jnp.maximum(ms[0], ms[1]),
                         jnp.maximum(ms[2], ms[3]))

        def _exp(i, ss):
            out = []
            for j in range(NCH):
                e = j * CH + i
                ee = jnp.exp(ltile[e, ts] - m1)
                etile[e, ts] = ee
                out.append(ss[j] + ee)
            return tuple(out)
        ss = lax.fori_loop(0, CH, _exp, (zero,) * NCH, unroll=4)
        s_full = (ss[0] + ss[1]) + (ss[2] + ss[3])

        am_prev = jnp.full((L,), -1, jnp.int32)
        for k in range(TOP_K):
            def _am(i, mv):
                out = []
                for j in range(NCH):
                    m, am = mv[2 * j], mv[2 * j + 1]
                    e = j * CH + i
                    ev = jnp.full((L,), e, jnp.int32)
                    v = ltile[e, ts]
                    hit = jnp.where(am_prev == ev, 1.0, 0.0)
                    v = v - hit * jnp.float32(1e30)
                    ltile[e, ts] = v
                    gt = v > m
                    out.extend((jnp.where(gt, v, m), jnp.where(gt, ev, am)))
                return tuple(out)
            z32 = jnp.zeros((L,), jnp.int32)
            mv = lax.fori_loop(0, CH, _am,
                               (neginf, z32) * NCH, unroll=4)
            m, am = mv[0], mv[1]
            for j in range(1, NCH):
                gt = mv[2 * j] > m
                m = jnp.where(gt, mv[2 * j], m)
                am = jnp.where(gt, mv[2 * j + 1], am)
            itile[k, ts] = am
            am_prev = am

        def _sel(i, ss2):
            out = []
            for j in range(NCH):
                e = j * CH + i
                ev = jnp.full((L,), e, jnp.int32)
                knocked = jnp.where(
                    ltile[e, ts] < jnp.float32(-1e29), 1.0, 0.0)
                last = jnp.where(am_prev == ev, 1.0, 0.0)
                up = jnp.maximum(knocked, last) * etile[e, ts]
                ptile[e, ts] = up
                out.append(ss2[j] + up)
            return tuple(out)
        ss2 = lax.fori_loop(0, CH, _sel, (zero,) * NCH, unroll=4)
        s_sel = (ss2[0] + ss2[1]) + (ss2[2] + ss2[3])

        inv_sel = 1.0 / s_sel
        inv_full = 1.0 / s_full

        def _fin(i, c2):
            for j in range(NCH):
                e = j * CH + i
                up = ptile[e, ts]
                ev = etile[e, ts]
                ptile[e, ts] = up * inv_sel
                accp[e, :] = accp[e, :] + ev * inv_full
                accm[e, :] = accm[e, :] + jnp.where(up > 0.0, 1.0, 0.0)
            return c2
        lax.fori_loop(0, CH, _fin, 0, unroll=4)
        return c

    lax.fori_loop(0, ng, _group, 0)

    pltpu.sync_copy(ptile, probs_hbm.at[:, pl.ds(base, tpw)])
    pltpu.sync_copy(itile, idx_hbm.at[:, pl.ds(base, tpw)])
    pltpu.sync_copy(accp, stats_hbm.at[0, wid])
    pltpu.sync_copy(accm, stats_hbm.at[1, wid])


def _loss_body(total_tokens, stats_ref, loss_ref):
    ps = jnp.sum(stats_ref[0], axis=(0, 2))
    ms = jnp.sum(stats_ref[1], axis=(0, 2))
    scale = jnp.float32(N_EXPERTS) / jnp.float32(total_tokens * total_tokens)
    loss_ref[0, 0] = scale * jnp.sum(ps * ms)


def kernel(x, W_gate, W_noise):
    del W_noise
    B, S, D = x.shape
    T = B * S
    xf = x.reshape(T, D)
    nsteps = T // BT
    tpw = T // NW
    ng = tpw // L

    logits_t = pl.pallas_call(
        _matmul_body,
        grid=(nsteps,),
        in_specs=[
            pl.BlockSpec((BT, D), lambda i: (i, 0)),
            pl.BlockSpec((N_EXPERTS, D), lambda i: (0, 0)),
        ],
        out_specs=pl.BlockSpec((N_EXPERTS, BT), lambda i: (0, i)),
        out_shape=jax.ShapeDtypeStruct((N_EXPERTS, T), jnp.float32),
        compiler_params=pltpu.CompilerParams(
            dimension_semantics=("arbitrary",)),
    )(xf, W_gate)

    mesh = plsc.VectorSubcoreMesh(core_axis_name="c", subcore_axis_name="s")
    route = functools.partial(
        pl.kernel,
        mesh=mesh,
        out_type=[
            jax.ShapeDtypeStruct((N_EXPERTS, T), jnp.float32),
            jax.ShapeDtypeStruct((TOP_K, T), jnp.int32),
            jax.ShapeDtypeStruct((2, NW, N_EXPERTS, L), jnp.float32),
        ],
        scratch_types=[
            pltpu.VMEM((N_EXPERTS, tpw), jnp.float32),
            pltpu.VMEM((N_EXPERTS, tpw), jnp.float32),
            pltpu.VMEM((N_EXPERTS, tpw), jnp.float32),
            pltpu.VMEM((TOP_K, tpw), jnp.int32),
            pltpu.VMEM((N_EXPERTS, L), jnp.float32),
            pltpu.VMEM((N_EXPERTS, L), jnp.float32),
        ],
    )(functools.partial(_route_body, tpw, ng))
    probs_t, idx_t, stats = route(logits_t)

    loss = pl.pallas_call(
        functools.partial(_loss_body, T),
        in_specs=[pl.BlockSpec((2, NW, N_EXPERTS, L), lambda: (0, 0, 0, 0))],
        out_specs=pl.BlockSpec(memory_space=pltpu.SMEM),
        out_shape=jax.ShapeDtypeStruct((1, 1), jnp.float32),
    )(stats)

    return (probs_t.T.reshape(B, S, N_EXPERTS),
            idx_t.T.reshape(B, S, TOP_K), loss.reshape(()))


# speedup vs baseline: 3.6711x; 3.6711x over previous
import functools

import jax
import jax.numpy as jnp
from jax.experimental import pallas as pl
from jax.experimental.pallas import tpu as pltpu

EMBED_DIM = 4096
N_EXPERTS = 64
TOP_K = 8
BT = 1024


def _router_body(nsteps, total_tokens, x_ref, w_ref, probs_ref, idx_ref,
                 loss_ref, accp_ref, accm_ref):
    i = pl.program_id(0)
    logits = jax.lax.dot_general(
        w_ref[...], x_ref[...], (((1,), (1,)), ((), ())),
        preferred_element_type=jnp.float32)

    iota_e = jax.lax.broadcasted_iota(jnp.int32, (N_EXPERTS, BT), 0)
    l = logits
    sel = jnp.zeros((N_EXPERTS, BT), jnp.bool_)
    ids = []
    m1 = None
    for k in range(TOP_K):
        m = jnp.max(l, axis=0, keepdims=True)
        if k == 0:
            m1 = m
        cand = jnp.where(l == m, iota_e, N_EXPERTS)
        idx = jnp.min(cand, axis=0, keepdims=True)
        pick = iota_e == idx
        sel = jnp.logical_or(sel, pick)
        ids.append(idx)
        l = jnp.where(pick, -jnp.inf, l)
    idx_ref[...] = jnp.concatenate(ids, axis=0)

    e = jnp.exp(logits - m1)
    e_sel = jnp.where(sel, e, 0.0)
    probs_ref[...] = e_sel / jnp.sum(e_sel, axis=0, keepdims=True)

    pfull = e / jnp.sum(e, axis=0, keepdims=True)
    self_f = sel.astype(jnp.float32)

    @pl.when(i == 0)
    def _init():
        accp_ref[...] = pfull
        accm_ref[...] = self_f

    @pl.when(i > 0)
    def _acc():
        accp_ref[...] += pfull
        accm_ref[...] += self_f

    @pl.when(i == nsteps - 1)
    def _fin():
        ps = jnp.sum(accp_ref[...], axis=1)
        ms = jnp.sum(accm_ref[...], axis=1)
        scale = jnp.float32(N_EXPERTS) / jnp.float32(total_tokens * total_tokens)
        loss_ref[0, 0] = scale * jnp.sum(ps * ms)


def kernel(x, W_gate, W_noise):
    del W_noise
    B, S, D = x.shape
    T = B * S
    xf = x.reshape(T, D)
    nsteps = T // BT

    probs_t, idx_t, loss = pl.pallas_call(
        functools.partial(_router_body, nsteps, T),
        grid=(nsteps,),
        in_specs=[
            pl.BlockSpec((BT, D), lambda i: (i, 0)),
            pl.BlockSpec((N_EXPERTS, D), lambda i: (0, 0)),
        ],
        out_specs=[
            pl.BlockSpec((N_EXPERTS, BT), lambda i: (0, i)),
            pl.BlockSpec((TOP_K, BT), lambda i: (0, i)),
            pl.BlockSpec(memory_space=pltpu.SMEM),
        ],
        out_shape=[
            jax.ShapeDtypeStruct((N_EXPERTS, T), jnp.float32),
            jax.ShapeDtypeStruct((TOP_K, T), jnp.int32),
            jax.ShapeDtypeStruct((1, 1), jnp.float32),
        ],
        scratch_shapes=[
            pltpu.VMEM((N_EXPERTS, BT), jnp.float32),
            pltpu.VMEM((N_EXPERTS, BT), jnp.float32),
        ],
        compiler_params=pltpu.CompilerParams(
            dimension_semantics=("arbitrary",)),
    )(xf, W_gate)

    return (probs_t.T.reshape(B, S, N_EXPERTS),
            idx_t.T.reshape(B, S, TOP_K), loss.reshape(()))
